# baseline (speedup 1.0000x reference)
import functools
import math

import jax
import jax.numpy as jnp
from jax import lax
from jax.experimental import pallas as pl
from jax.experimental.pallas import tpu as pltpu
from jax.experimental.pallas import tpu_sc as plsc

EMBED = 256
NH = 8
NP = 4
HD = EMBED // NH
NCORN = 4
NSAMP = NH * NP * NCORN
NC, NS, LANES = 2, 16, 16
NW = NC * NS


def _matmul_bias_body(x_ref, w_ref, b_ref, o_ref, *, out_dtype):
    o_ref[...] = (
        jnp.dot(x_ref[...], w_ref[...], preferred_element_type=jnp.float32)
        + b_ref[...]
    ).astype(out_dtype)


def _matmul_bias(x, w, b, blk, out_dtype=jnp.float32):
    m, k = x.shape
    n = w.shape[1]
    return pl.pallas_call(
        functools.partial(_matmul_bias_body, out_dtype=out_dtype),
        grid=(m // blk,),
        in_specs=[
            pl.BlockSpec((blk, k), lambda i: (i, 0)),
            pl.BlockSpec((k, n), lambda i: (0, 0)),
            pl.BlockSpec((1, n), lambda i: (0, 0)),
        ],
        out_specs=pl.BlockSpec((blk, n), lambda i: (i, 0)),
        out_shape=jax.ShapeDtypeStruct((m, n), out_dtype),
    )(x, w, b)


def _prep_body(q_ref, rp_ref, wox_ref, woy_ref, wa_ref, bo_ref, ba_ref,
               s_ref, idx_ref, wgt_ref, *, Hs, Ws, HW, TQ):
    b = pl.program_id(0)
    q = q_ref[0]
    rp = rp_ref[0]
    rpx = rp[:, 0:1]
    rpy = rp[:, 1:2]
    offx = jnp.dot(q, wox_ref[...], preferred_element_type=jnp.float32) + bo_ref[0:1]
    offy = jnp.dot(q, woy_ref[...], preferred_element_type=jnp.float32) + bo_ref[1:2]
    lg = jnp.dot(q, wa_ref[...], preferred_element_type=jnp.float32) + ba_ref[...]
    m = jnp.max(lg, axis=1, keepdims=True)
    e = jnp.exp(lg - m)
    aw = e / jnp.dot(e, s_ref[...], preferred_element_type=jnp.float32)
    x = rpx * Ws + offx - 0.5
    y = rpy * Hs + offy - 0.5
    x0 = jnp.floor(x)
    fx = x - x0
    x1 = x0 + 1.0
    y0 = jnp.floor(y)
    fy = y - y0
    y1 = y0 + 1.0
    xs = jnp.concatenate([x0, x1, x0, x1], axis=1)
    ys = jnp.concatenate([y0, y0, y1, y1], axis=1)
    wxs = jnp.concatenate([1.0 - fx, fx, 1.0 - fx, fx], axis=1)
    wys = jnp.concatenate([1.0 - fy, 1.0 - fy, fy, fy], axis=1)
    aw4 = jnp.concatenate([aw, aw, aw, aw], axis=1)
    valid = (xs >= 0) & (xs <= Ws - 1) & (ys >= 0) & (ys <= Hs - 1)
    wgt_ref[0] = jnp.where(valid, aw4 * wxs * wys, 0.0)
    ixc = jnp.clip(xs, 0, Ws - 1).astype(jnp.int32)
    iyc = jnp.clip(ys, 0, Hs - 1).astype(jnp.int32)
    hlane = (lax.broadcasted_iota(jnp.int32, (TQ, NSAMP), 1) % (NH * NP)) // NP
    idx_ref[0] = b * (HW * NH) + (iyc * Ws + ixc) * NH + hlane


def _sampling_prep(query, rp, wox, woy, wa, bo2, ba2, smat, Hs, Ws, TQ):
    bs, nq, _ = query.shape
    HW = Hs * Ws
    body = functools.partial(_prep_body, Hs=Hs, Ws=Ws, HW=HW, TQ=TQ)
    hp = NH * NP
    return pl.pallas_call(
        body,
        grid=(bs, nq // TQ),
        in_specs=[
            pl.BlockSpec((1, TQ, EMBED), lambda b, t: (b, t, 0)),
            pl.BlockSpec((1, TQ, 2), lambda b, t: (b, t, 0)),
            pl.BlockSpec((EMBED, hp), lambda b, t: (0, 0)),
            pl.BlockSpec((EMBED, hp), lambda b, t: (0, 0)),
            pl.BlockSpec((EMBED, hp), lambda b, t: (0, 0)),
            pl.BlockSpec((2, hp), lambda b, t: (0, 0)),
            pl.BlockSpec((1, hp), lambda b, t: (0, 0)),
            pl.BlockSpec((hp, hp), lambda b, t: (0, 0)),
        ],
        out_specs=[
            pl.BlockSpec((1, TQ, NSAMP), lambda b, t: (b, t, 0)),
            pl.BlockSpec((1, TQ, NSAMP), lambda b, t: (b, t, 0)),
        ],
        out_shape=[
            jax.ShapeDtypeStruct((bs, nq, NSAMP), jnp.int32),
            jax.ShapeDtypeStruct((bs, nq, NSAMP), jnp.float32),
        ],
    )(query, rp, wox, woy, wa, bo2, ba2, smat)


def _bcast_lane(vec, s):
    return jnp.broadcast_to(lax.slice(vec, (s,), (s + 1,)), (LANES,))


def _sc_sample_combine(table, idx2, wgt2):
    BQ = idx2.shape[0]
    QW = BQ // NW
    C = 16
    NCH = QW // C
    GR = C * NSAMP // 128
    mesh = plsc.VectorSubcoreMesh(core_axis_name="c", subcore_axis_name="s")

    @functools.partial(
        pl.kernel,
        mesh=mesh,
        compiler_params=pltpu.CompilerParams(use_tc_tiling_on_sc=False,
                                             needs_layout_passes=False),
        out_type=jax.ShapeDtypeStruct((BQ, EMBED), jnp.float32),
        scratch_types=[
            pltpu.VMEM((C, NSAMP), jnp.int32),
            pltpu.VMEM((C, NSAMP), jnp.int32),
            pltpu.VMEM((C, NSAMP), jnp.float32),
            pltpu.VMEM((C, NSAMP), jnp.float32),
            pltpu.VMEM((C * NSAMP, HD // 2), jnp.int32),
            pltpu.VMEM((C * NSAMP, HD // 2), jnp.int32),
            pltpu.VMEM((C, EMBED), jnp.float32),
            pltpu.SemaphoreType.DMA,
            pltpu.SemaphoreType.DMA,
        ],
    )
    def k(table_r, idx_r, wgt_r, out_r, idx_v0, idx_v1, wgt_v0, wgt_v1,
          rows_v0, rows_v1, out_v, sem0, sem1):
        wid = lax.axis_index("s") * NC + lax.axis_index("c")
        base = wid * QW
        bufs = ((idx_v0, wgt_v0, rows_v0, sem0),
                (idx_v1, wgt_v1, rows_v1, sem1))

        def fire(jj, buf):
            idx_v, wgt_v, rows_v, sem = buf
            q0 = base + jj * C
            pltpu.sync_copy(idx_r.at[pl.ds(q0, C)], idx_v)
            pltpu.sync_copy(wgt_r.at[pl.ds(q0, C)], wgt_v)
            for g in range(GR):
                pltpu.async_copy(table_r.at[idx_v.at[g]],
                                 rows_v.at[pl.ds(g * 128, 128)], sem)

        def drain_compute_store(jj, buf):
            idx_v, wgt_v, rows_v, sem = buf
            for g in range(GR):
                pltpu.make_async_copy(table_r.at[idx_v.at[g]],
                                      rows_v.at[pl.ds(g * 128, 128)],
                                      sem).wait()

            def item_body(i, c2):
                isplat = jnp.broadcast_to(i, (LANES,)).astype(jnp.int32)
                for h in range(NH):
                    acc0 = jnp.zeros((LANES,), jnp.float32)
                    acc1 = jnp.zeros((LANES,), jnp.float32)
                    for c in range(NCORN):
                        for p in range(NP):
                            s = c * (NH * NP) + h * NP + p
                            ws = plsc.load_gather(
                                wgt_v,
                                [isplat,
                                 jnp.full((LANES,), s, jnp.int32)])
                            ri = rows_v[i * NSAMP + s]
                            lo = plsc.bitcast(
                                lax.shift_left(ri, 16), jnp.float32)
                            hi = plsc.bitcast(
                                ri & jnp.int32(-65536), jnp.float32)
                            acc0 = acc0 + ws * lo
                            acc1 = acc1 + ws * hi
                    out_v[i, pl.ds(h * HD, LANES)] = acc0
                    out_v[i, pl.ds(h * HD + LANES, LANES)] = acc1
                return c2

            lax.fori_loop(0, C, item_body, 0)
            pltpu.sync_copy(out_v, out_r.at[pl.ds(base + jj * C, C)])

        fire(0, bufs[0])

        def pair_body(t, carry):
            j0 = 2 * t
            fire(j0 + 1, bufs[1])
            drain_compute_store(j0, bufs[0])

            @pl.when(t < NCH // 2 - 1)
            def _():
                fire(j0 + 2, bufs[0])

            drain_compute_store(j0 + 1, bufs[1])
            return carry

        lax.fori_loop(0, NCH // 2, pair_body, 0)

    return k(table, idx2, wgt2)


def kernel(query, reference_points, value, Wv, bv, Wo, bo, Wa, ba, Wout,
           bout, spatial_shape):
    bs, nq, _ = query.shape
    HW = value.shape[1]
    Hs = int(math.isqrt(HW))
    Ws = HW // Hs

    v = _matmul_bias(value.reshape(bs * HW, EMBED), Wv.T,
                     bv.reshape(1, EMBED), blk=1024, out_dtype=jnp.bfloat16)
    table = lax.bitcast_convert_type(
        v.reshape(bs * HW * NH, HD // 2, 2), jnp.int32)

    hp = NH * NP
    Wo4 = Wo.reshape(hp, 2, EMBED)
    wox = Wo4[:, 0, :].T
    woy = Wo4[:, 1, :].T
    bo2 = bo.reshape(hp, 2).T
    wa = Wa.T
    ba2 = ba.reshape(1, hp)
    gid = jnp.arange(hp, dtype=jnp.int32) // NP
    smat = (gid[:, None] == gid[None, :]).astype(jnp.float32)
    idx, wgt = _sampling_prep(query, reference_points, wox, woy, wa,
                              bo2, ba2, smat, Hs, Ws, TQ=512)

    headout = _sc_sample_combine(table, idx.reshape(bs * nq, NSAMP),
                                 wgt.reshape(bs * nq, NSAMP))

    ar = jnp.arange(EMBED, dtype=jnp.int32)
    perm = (ar // HD) * HD + 2 * (ar % LANES) + (ar % HD) // LANES
    out = _matmul_bias(headout, Wout.T[perm], bout.reshape(1, EMBED),
                       blk=1024)
    return out.reshape(bs, nq, EMBED)


# speedup vs baseline: 30.4655x; 30.4655x over previous
import functools
import math

import jax
import jax.numpy as jnp
from jax import lax
from jax.experimental import pallas as pl
from jax.experimental.pallas import tpu as pltpu
from jax.experimental.pallas import tpu_sc as plsc

EMBED = 256
NH = 8
NP = 4
HD = EMBED // NH
NCORN = 4
NSAMP = NH * NP * NCORN
NC, NS, LANES = 2, 16, 16
NW = NC * NS


def _matmul_bias_body(x_ref, w_ref, b_ref, o_ref, *, out_dtype):
    o_ref[...] = (
        jnp.dot(x_ref[...], w_ref[...], preferred_element_type=jnp.float32)
        + b_ref[...]
    ).astype(out_dtype)


def _matmul_bias(x, w, b, blk, out_dtype=jnp.float32):
    m, k = x.shape
    n = w.shape[1]
    return pl.pallas_call(
        functools.partial(_matmul_bias_body, out_dtype=out_dtype),
        grid=(m // blk,),
        in_specs=[
            pl.BlockSpec((blk, k), lambda i: (i, 0)),
            pl.BlockSpec((k, n), lambda i: (0, 0)),
            pl.BlockSpec((1, n), lambda i: (0, 0)),
        ],
        out_specs=pl.BlockSpec((blk, n), lambda i: (i, 0)),
        out_shape=jax.ShapeDtypeStruct((m, n), out_dtype),
    )(x, w, b)


def _prep_body(q_ref, rp_ref, wox_ref, woy_ref, wa_ref, bo_ref, ba_ref,
               s_ref, idx_ref, wgt_ref, *, Hs, Ws, HW, TQ):
    b = pl.program_id(0)
    q = q_ref[0]
    rp = rp_ref[0]
    rpx = rp[:, 0:1]
    rpy = rp[:, 1:2]
    offx = jnp.dot(q, wox_ref[...], preferred_element_type=jnp.float32) + bo_ref[0:1]
    offy = jnp.dot(q, woy_ref[...], preferred_element_type=jnp.float32) + bo_ref[1:2]
    lg = jnp.dot(q, wa_ref[...], preferred_element_type=jnp.float32) + ba_ref[...]
    m = jnp.max(lg, axis=1, keepdims=True)
    e = jnp.exp(lg - m)
    aw = e / jnp.dot(e, s_ref[...], preferred_element_type=jnp.float32)
    x = rpx * Ws + offx - 0.5
    y = rpy * Hs + offy - 0.5
    x0 = jnp.floor(x)
    fx = x - x0
    x1 = x0 + 1.0
    y0 = jnp.floor(y)
    fy = y - y0
    y1 = y0 + 1.0
    xs = jnp.concatenate([x0, x1, x0, x1], axis=1)
    ys = jnp.concatenate([y0, y0, y1, y1], axis=1)
    wxs = jnp.concatenate([1.0 - fx, fx, 1.0 - fx, fx], axis=1)
    wys = jnp.concatenate([1.0 - fy, 1.0 - fy, fy, fy], axis=1)
    aw4 = jnp.concatenate([aw, aw, aw, aw], axis=1)
    valid = (xs >= 0) & (xs <= Ws - 1) & (ys >= 0) & (ys <= Hs - 1)
    wgt_ref[0] = jnp.where(valid, aw4 * wxs * wys, 0.0)
    ixc = jnp.clip(xs, 0, Ws - 1).astype(jnp.int32)
    iyc = jnp.clip(ys, 0, Hs - 1).astype(jnp.int32)
    hlane = (lax.broadcasted_iota(jnp.int32, (TQ, NSAMP), 1) % (NH * NP)) // NP
    idx_ref[0] = b * (HW * NH) + (iyc * Ws + ixc) * NH + hlane


def _sampling_prep(query, rp, wox, woy, wa, bo2, ba2, smat, Hs, Ws, TQ):
    bs, nq, _ = query.shape
    HW = Hs * Ws
    body = functools.partial(_prep_body, Hs=Hs, Ws=Ws, HW=HW, TQ=TQ)
    hp = NH * NP
    return pl.pallas_call(
        body,
        grid=(bs, nq // TQ),
        in_specs=[
            pl.BlockSpec((1, TQ, EMBED), lambda b, t: (b, t, 0)),
            pl.BlockSpec((1, TQ, 2), lambda b, t: (b, t, 0)),
            pl.BlockSpec((EMBED, hp), lambda b, t: (0, 0)),
            pl.BlockSpec((EMBED, hp), lambda b, t: (0, 0)),
            pl.BlockSpec((EMBED, hp), lambda b, t: (0, 0)),
            pl.BlockSpec((2, hp), lambda b, t: (0, 0)),
            pl.BlockSpec((1, hp), lambda b, t: (0, 0)),
            pl.BlockSpec((hp, hp), lambda b, t: (0, 0)),
        ],
        out_specs=[
            pl.BlockSpec((1, TQ, NSAMP), lambda b, t: (b, t, 0)),
            pl.BlockSpec((1, TQ, NSAMP), lambda b, t: (b, t, 0)),
        ],
        out_shape=[
            jax.ShapeDtypeStruct((bs, nq, NSAMP), jnp.int32),
            jax.ShapeDtypeStruct((bs, nq, NSAMP), jnp.float32),
        ],
    )(query, rp, wox, woy, wa, bo2, ba2, smat)


def _bcast_lane(vec, s):
    return jnp.broadcast_to(lax.slice(vec, (s,), (s + 1,)), (LANES,))


def _sc_sample_combine(table, idx2, wgt2):
    BQ = idx2.shape[0]
    QW = BQ // NW
    C = 8
    NCH = QW // C
    GR = C * NSAMP // 128
    mesh = plsc.VectorSubcoreMesh(core_axis_name="c", subcore_axis_name="s")

    @functools.partial(
        pl.kernel,
        mesh=mesh,
        compiler_params=pltpu.CompilerParams(use_tc_tiling_on_sc=False,
                                             needs_layout_passes=False),
        out_type=jax.ShapeDtypeStruct((BQ, EMBED), jnp.float32),
        scratch_types=[
            pltpu.VMEM((C, NSAMP), jnp.int32),
            pltpu.VMEM((C, NSAMP), jnp.int32),
            pltpu.VMEM((C, NSAMP), jnp.float32),
            pltpu.VMEM((C, NSAMP), jnp.float32),
            pltpu.VMEM((C * NSAMP, HD), jnp.float32),
            pltpu.VMEM((C * NSAMP, HD), jnp.float32),
            pltpu.VMEM((C, EMBED), jnp.float32),
            pltpu.SemaphoreType.DMA,
            pltpu.SemaphoreType.DMA,
        ],
    )
    def k(table_r, idx_r, wgt_r, out_r, idx_v0, idx_v1, wgt_v0, wgt_v1,
          rows_v0, rows_v1, out_v, sem0, sem1):
        wid = lax.axis_index("s") * NC + lax.axis_index("c")
        base = wid * QW
        bufs = ((idx_v0, wgt_v0, rows_v0, sem0),
                (idx_v1, wgt_v1, rows_v1, sem1))

        def fire(jj, buf):
            idx_v, wgt_v, rows_v, sem = buf
            q0 = base + jj * C
            pltpu.sync_copy(idx_r.at[pl.ds(q0, C)], idx_v)
            pltpu.sync_copy(wgt_r.at[pl.ds(q0, C)], wgt_v)
            for g in range(GR):
                pltpu.async_copy(table_r.at[idx_v.at[g]],
                                 rows_v.at[pl.ds(g * 128, 128)], sem)

        def drain_compute_store(jj, buf):
            idx_v, wgt_v, rows_v, sem = buf
            for g in range(GR):
                pltpu.make_async_copy(table_r.at[idx_v.at[g]],
                                      rows_v.at[pl.ds(g * 128, 128)],
                                      sem).wait()

            def item_body(i, c2):
                isplat = jnp.broadcast_to(i, (LANES,)).astype(jnp.int32)
                for h in range(NH):
                    acc0 = jnp.zeros((LANES,), jnp.float32)
                    acc1 = jnp.zeros((LANES,), jnp.float32)
                    for c in range(NCORN):
                        for p in range(NP):
                            s = c * (NH * NP) + h * NP + p
                            ws = plsc.load_gather(
                                wgt_v,
                                [isplat,
                                 jnp.full((LANES,), s, jnp.int32)])
                            r = i * NSAMP + s
                            acc0 = acc0 + ws * rows_v[r, pl.ds(0, LANES)]
                            acc1 = acc1 + ws * rows_v[r, pl.ds(LANES, LANES)]
                    out_v[i, pl.ds(h * HD, LANES)] = acc0
                    out_v[i, pl.ds(h * HD + LANES, LANES)] = acc1
                return c2

            lax.fori_loop(0, C, item_body, 0)
            pltpu.sync_copy(out_v, out_r.at[pl.ds(base + jj * C, C)])

        fire(0, bufs[0])

        def pair_body(t, carry):
            j0 = 2 * t
            fire(j0 + 1, bufs[1])
            drain_compute_store(j0, bufs[0])

            @pl.when(t < NCH // 2 - 1)
            def _():
                fire(j0 + 2, bufs[0])

            drain_compute_store(j0 + 1, bufs[1])
            return carry

        lax.fori_loop(0, NCH // 2, pair_body, 0)

    return k(table, idx2, wgt2)


def kernel(query, reference_points, value, Wv, bv, Wo, bo, Wa, ba, Wout,
           bout, spatial_shape):
    bs, nq, _ = query.shape
    HW = value.shape[1]
    Hs = int(math.isqrt(HW))
    Ws = HW // Hs

    v = _matmul_bias(value.reshape(bs * HW, EMBED), Wv.T,
                     bv.reshape(1, EMBED), blk=1024)
    table = v.reshape(bs * HW * NH, HD)

    hp = NH * NP
    Wo4 = Wo.reshape(hp, 2, EMBED)
    wox = Wo4[:, 0, :].T
    woy = Wo4[:, 1, :].T
    bo2 = bo.reshape(hp, 2).T
    wa = Wa.T
    ba2 = ba.reshape(1, hp)
    gid = jnp.arange(hp, dtype=jnp.int32) // NP
    smat = (gid[:, None] == gid[None, :]).astype(jnp.float32)
    idx, wgt = _sampling_prep(query, reference_points, wox, woy, wa,
                              bo2, ba2, smat, Hs, Ws, TQ=512)

    headout = _sc_sample_combine(table, idx.reshape(bs * nq, NSAMP),
                                 wgt.reshape(bs * nq, NSAMP))

    out = _matmul_bias(headout, Wout.T, bout.reshape(1, EMBED), blk=1024)
    return out.reshape(bs, nq, EMBED)


# speedup vs baseline: 36.6606x; 1.2033x over previous
import functools
import math

import jax
import jax.numpy as jnp
from jax import lax
from jax.experimental import pallas as pl
from jax.experimental.pallas import tpu as pltpu
from jax.experimental.pallas import tpu_sc as plsc

EMBED = 256
NH = 8
NP = 4
HD = EMBED // NH
NCORN = 4
NSAMP = NH * NP * NCORN
NC, NS, LANES = 2, 16, 16
NW = NC * NS


def _matmul_bias_body(x_ref, w_ref, b_ref, o_ref, *, out_dtype):
    o_ref[...] = (
        jnp.dot(x_ref[...], w_ref[...], preferred_element_type=jnp.float32)
        + b_ref[...]
    ).astype(out_dtype)


def _vproj_pack_body(x_ref, wlo_ref, whi_ref, blo_ref, bhi_ref, o_ref):
    def half(w_ref, b_ref):
        y = (jnp.dot(x_ref[...], w_ref[...],
                     preferred_element_type=jnp.float32) + b_ref[...])
        return lax.bitcast_convert_type(
            y.astype(jnp.bfloat16).astype(jnp.float32), jnp.int32)

    o_ref[...] = (lax.shift_right_logical(half(wlo_ref, blo_ref), 16)
                  | (half(whi_ref, bhi_ref) & jnp.int32(-65536)))


def _vproj_pack(x, wlo, whi, blo, bhi, blk):
    m, k = x.shape
    n = wlo.shape[1]
    return pl.pallas_call(
        _vproj_pack_body,
        grid=(m // blk,),
        in_specs=[
            pl.BlockSpec((blk, k), lambda i: (i, 0)),
            pl.BlockSpec((k, n), lambda i: (0, 0)),
            pl.BlockSpec((k, n), lambda i: (0, 0)),
            pl.BlockSpec((1, n), lambda i: (0, 0)),
            pl.BlockSpec((1, n), lambda i: (0, 0)),
        ],
        out_specs=pl.BlockSpec((blk, n), lambda i: (i, 0)),
        out_shape=jax.ShapeDtypeStruct((m, n), jnp.int32),
    )(x, wlo, whi, blo, bhi)


def _matmul_bias(x, w, b, blk, out_dtype=jnp.float32):
    m, k = x.shape
    n = w.shape[1]
    return pl.pallas_call(
        functools.partial(_matmul_bias_body, out_dtype=out_dtype),
        grid=(m // blk,),
        in_specs=[
            pl.BlockSpec((blk, k), lambda i: (i, 0)),
            pl.BlockSpec((k, n), lambda i: (0, 0)),
            pl.BlockSpec((1, n), lambda i: (0, 0)),
        ],
        out_specs=pl.BlockSpec((blk, n), lambda i: (i, 0)),
        out_shape=jax.ShapeDtypeStruct((m, n), out_dtype),
    )(x, w, b)


def _prep_body(q_ref, rp_ref, wox_ref, woy_ref, wa_ref, bo_ref, ba_ref,
               s_ref, idx_ref, wgt_ref, *, Hs, Ws, HW, TQ):
    b = pl.program_id(0)
    q = q_ref[0]
    rp = rp_ref[0]
    rpx = rp[:, 0:1]
    rpy = rp[:, 1:2]
    offx = jnp.dot(q, wox_ref[...], preferred_element_type=jnp.float32) + bo_ref[0:1]
    offy = jnp.dot(q, woy_ref[...], preferred_element_type=jnp.float32) + bo_ref[1:2]
    lg = jnp.dot(q, wa_ref[...], preferred_element_type=jnp.float32) + ba_ref[...]
    m = jnp.max(lg, axis=1, keepdims=True)
    e = jnp.exp(lg - m)
    aw = e / jnp.dot(e, s_ref[...], preferred_element_type=jnp.float32)
    x = rpx * Ws + offx - 0.5
    y = rpy * Hs + offy - 0.5
    x0 = jnp.floor(x)
    fx = x - x0
    x1 = x0 + 1.0
    y0 = jnp.floor(y)
    fy = y - y0
    y1 = y0 + 1.0
    xs = jnp.concatenate([x0, x1, x0, x1], axis=1)
    ys = jnp.concatenate([y0, y0, y1, y1], axis=1)
    wxs = jnp.concatenate([1.0 - fx, fx, 1.0 - fx, fx], axis=1)
    wys = jnp.concatenate([1.0 - fy, 1.0 - fy, fy, fy], axis=1)
    aw4 = jnp.concatenate([aw, aw, aw, aw], axis=1)
    valid = (xs >= 0) & (xs <= Ws - 1) & (ys >= 0) & (ys <= Hs - 1)
    wgt_ref[0] = jnp.where(valid, aw4 * wxs * wys, 0.0)
    ixc = jnp.clip(xs, 0, Ws - 1).astype(jnp.int32)
    iyc = jnp.clip(ys, 0, Hs - 1).astype(jnp.int32)
    hlane = (lax.broadcasted_iota(jnp.int32, (TQ, NSAMP), 1) % (NH * NP)) // NP
    idx_ref[0] = b * (HW * NH) + (iyc * Ws + ixc) * NH + hlane


def _sampling_prep(query, rp, wox, woy, wa, bo2, ba2, smat, Hs, Ws, TQ):
    bs, nq, _ = query.shape
    HW = Hs * Ws
    body = functools.partial(_prep_body, Hs=Hs, Ws=Ws, HW=HW, TQ=TQ)
    hp = NH * NP
    return pl.pallas_call(
        body,
        grid=(bs, nq // TQ),
        in_specs=[
            pl.BlockSpec((1, TQ, EMBED), lambda b, t: (b, t, 0)),
            pl.BlockSpec((1, TQ, 2), lambda b, t: (b, t, 0)),
            pl.BlockSpec((EMBED, hp), lambda b, t: (0, 0)),
            pl.BlockSpec((EMBED, hp), lambda b, t: (0, 0)),
            pl.BlockSpec((EMBED, hp), lambda b, t: (0, 0)),
            pl.BlockSpec((2, hp), lambda b, t: (0, 0)),
            pl.BlockSpec((1, hp), lambda b, t: (0, 0)),
            pl.BlockSpec((hp, hp), lambda b, t: (0, 0)),
        ],
        out_specs=[
            pl.BlockSpec((1, TQ, NSAMP), lambda b, t: (b, t, 0)),
            pl.BlockSpec((1, TQ, NSAMP), lambda b, t: (b, t, 0)),
        ],
        out_shape=[
            jax.ShapeDtypeStruct((bs, nq, NSAMP), jnp.int32),
            jax.ShapeDtypeStruct((bs, nq, NSAMP), jnp.float32),
        ],
    )(query, rp, wox, woy, wa, bo2, ba2, smat)


def _bcast_lane(vec, s):
    return jnp.broadcast_to(lax.slice(vec, (s,), (s + 1,)), (LANES,))


def _sc_sample_combine(table, idx2, wgt2):
    BQ = idx2.shape[0]
    QW = BQ // NW
    C = 16
    NCH = QW // C
    GR = C * NSAMP // 128
    mesh = plsc.VectorSubcoreMesh(core_axis_name="c", subcore_axis_name="s")

    @functools.partial(
        pl.kernel,
        mesh=mesh,
        compiler_params=pltpu.CompilerParams(use_tc_tiling_on_sc=False,
                                             needs_layout_passes=False),
        out_type=jax.ShapeDtypeStruct((BQ, EMBED), jnp.float32),
        scratch_types=[
            pltpu.VMEM((C, NSAMP), jnp.int32),
            pltpu.VMEM((C, NSAMP), jnp.int32),
            pltpu.VMEM((C, NSAMP), jnp.float32),
            pltpu.VMEM((C, NSAMP), jnp.float32),
            pltpu.VMEM((C * NSAMP, HD // 2), jnp.int32),
            pltpu.VMEM((C * NSAMP, HD // 2), jnp.int32),
            pltpu.VMEM((C, EMBED), jnp.float32),
            pltpu.SemaphoreType.DMA,
            pltpu.SemaphoreType.DMA,
        ],
    )
    def k(table_r, idx_r, wgt_r, out_r, idx_v0, idx_v1, wgt_v0, wgt_v1,
          rows_v0, rows_v1, out_v, sem0, sem1):
        wid = lax.axis_index("s") * NC + lax.axis_index("c")
        base = wid * QW
        bufs = ((idx_v0, wgt_v0, rows_v0, sem0),
                (idx_v1, wgt_v1, rows_v1, sem1))

        def fire(jj, buf):
            idx_v, wgt_v, rows_v, sem = buf
            q0 = base + jj * C
            pltpu.sync_copy(idx_r.at[pl.ds(q0, C)], idx_v)
            pltpu.sync_copy(wgt_r.at[pl.ds(q0, C)], wgt_v)
            for g in range(GR):
                pltpu.async_copy(table_r.at[idx_v.at[g]],
                                 rows_v.at[pl.ds(g * 128, 128)], sem)

        def drain_compute_store(jj, buf):
            idx_v, wgt_v, rows_v, sem = buf
            for g in range(GR):
                pltpu.make_async_copy(table_r.at[idx_v.at[g]],
                                      rows_v.at[pl.ds(g * 128, 128)],
                                      sem).wait()

            def item_body(i, c2):
                isplat = jnp.broadcast_to(i, (LANES,)).astype(jnp.int32)
                for h in range(NH):
                    acc0 = jnp.zeros((LANES,), jnp.float32)
                    acc1 = jnp.zeros((LANES,), jnp.float32)
                    for c in range(NCORN):
                        for p in range(NP):
                            s = c * (NH * NP) + h * NP + p
                            ws = plsc.load_gather(
                                wgt_v,
                                [isplat,
                                 jnp.full((LANES,), s, jnp.int32)])
                            ri = rows_v[i * NSAMP + s]
                            lo = plsc.bitcast(
                                lax.shift_left(ri, 16), jnp.float32)
                            hi = plsc.bitcast(
                                ri & jnp.int32(-65536), jnp.float32)
                            acc0 = acc0 + ws * lo
                            acc1 = acc1 + ws * hi
                    out_v[i, pl.ds(h * HD, LANES)] = acc0
                    out_v[i, pl.ds(h * HD + LANES, LANES)] = acc1
                return c2

            lax.fori_loop(0, C, item_body, 0)
            pltpu.sync_copy(out_v, out_r.at[pl.ds(base + jj * C, C)])

        fire(0, bufs[0])

        def pair_body(t, carry):
            j0 = 2 * t
            fire(j0 + 1, bufs[1])
            drain_compute_store(j0, bufs[0])

            @pl.when(t < NCH // 2 - 1)
            def _():
                fire(j0 + 2, bufs[0])

            drain_compute_store(j0 + 1, bufs[1])
            return carry

        lax.fori_loop(0, NCH // 2, pair_body, 0)

    return k(table, idx2, wgt2)


def kernel(query, reference_points, value, Wv, bv, Wo, bo, Wa, ba, Wout,
           bout, spatial_shape):
    bs, nq, _ = query.shape
    HW = value.shape[1]
    Hs = int(math.isqrt(HW))
    Ws = HW // Hs

    wvt = Wv.T
    ar = jnp.arange(EMBED // 2, dtype=jnp.int32)
    lo_cols = (ar // LANES) * HD + ar % LANES
    hi_cols = lo_cols + LANES
    vp = _vproj_pack(value.reshape(bs * HW, EMBED), wvt[:, lo_cols],
                     wvt[:, hi_cols], bv[lo_cols].reshape(1, -1),
                     bv[hi_cols].reshape(1, -1), blk=1024)
    table = vp.reshape(bs * HW * NH, HD // 2)

    hp = NH * NP
    Wo4 = Wo.reshape(hp, 2, EMBED)
    wox = Wo4[:, 0, :].T
    woy = Wo4[:, 1, :].T
    bo2 = bo.reshape(hp, 2).T
    wa = Wa.T
    ba2 = ba.reshape(1, hp)
    gid = jnp.arange(hp, dtype=jnp.int32) // NP
    smat = (gid[:, None] == gid[None, :]).astype(jnp.float32)
    idx, wgt = _sampling_prep(query, reference_points, wox, woy, wa,
                              bo2, ba2, smat, Hs, Ws, TQ=512)

    headout = _sc_sample_combine(table, idx.reshape(bs * nq, NSAMP),
                                 wgt.reshape(bs * nq, NSAMP))

    out = _matmul_bias(headout, Wout.T, bout.reshape(1, EMBED), blk=1024)
    return out.reshape(bs, nq, EMBED)


# speedup vs baseline: 37.1070x; 1.0122x over previous
import functools
import math

import jax
import jax.numpy as jnp
from jax import lax
from jax.experimental import pallas as pl
from jax.experimental.pallas import tpu as pltpu
from jax.experimental.pallas import tpu_sc as plsc

EMBED = 256
NH = 8
NP = 4
HD = EMBED // NH
NCORN = 4
NSAMP = NH * NP * NCORN
NC, NS, LANES = 2, 16, 16
NW = NC * NS


def _matmul_bias_body(x_ref, w_ref, b_ref, o_ref, *, out_dtype):
    o_ref[...] = (
        jnp.dot(x_ref[...], w_ref[...], preferred_element_type=jnp.float32)
        + b_ref[...]
    ).astype(out_dtype)


def _vproj_pack_body(x_ref, wlo_ref, whi_ref, blo_ref, bhi_ref, o_ref):
    def half(w_ref, b_ref):
        y = (jnp.dot(x_ref[...], w_ref[...],
                     preferred_element_type=jnp.float32) + b_ref[...])
        return lax.bitcast_convert_type(
            y.astype(jnp.bfloat16).astype(jnp.float32), jnp.int32)

    o_ref[...] = (lax.shift_right_logical(half(wlo_ref, blo_ref), 16)
                  | (half(whi_ref, bhi_ref) & jnp.int32(-65536)))


def _vproj_pack(x, wlo, whi, blo, bhi, blk):
    m, k = x.shape
    n = wlo.shape[1]
    return pl.pallas_call(
        _vproj_pack_body,
        grid=(m // blk,),
        in_specs=[
            pl.BlockSpec((blk, k), lambda i: (i, 0)),
            pl.BlockSpec((k, n), lambda i: (0, 0)),
            pl.BlockSpec((k, n), lambda i: (0, 0)),
            pl.BlockSpec((1, n), lambda i: (0, 0)),
            pl.BlockSpec((1, n), lambda i: (0, 0)),
        ],
        out_specs=pl.BlockSpec((blk, n), lambda i: (i, 0)),
        out_shape=jax.ShapeDtypeStruct((m, n), jnp.int32),
    )(x, wlo, whi, blo, bhi)


def _matmul_bias(x, w, b, blk, out_dtype=jnp.float32):
    m, k = x.shape
    n = w.shape[1]
    return pl.pallas_call(
        functools.partial(_matmul_bias_body, out_dtype=out_dtype),
        grid=(m // blk,),
        in_specs=[
            pl.BlockSpec((blk, k), lambda i: (i, 0)),
            pl.BlockSpec((k, n), lambda i: (0, 0)),
            pl.BlockSpec((1, n), lambda i: (0, 0)),
        ],
        out_specs=pl.BlockSpec((blk, n), lambda i: (i, 0)),
        out_shape=jax.ShapeDtypeStruct((m, n), out_dtype),
    )(x, w, b)


def _prep_body(q_ref, rp_ref, wox_ref, woy_ref, wa_ref, bo_ref, ba_ref,
               s_ref, idx_ref, wgt_ref, *, Hs, Ws, HW, TQ):
    b = pl.program_id(0)
    q = q_ref[0]
    rp = rp_ref[0]
    rpx = rp[:, 0:1]
    rpy = rp[:, 1:2]
    offx = jnp.dot(q, wox_ref[...], preferred_element_type=jnp.float32) + bo_ref[0:1]
    offy = jnp.dot(q, woy_ref[...], preferred_element_type=jnp.float32) + bo_ref[1:2]
    lg = jnp.dot(q, wa_ref[...], preferred_element_type=jnp.float32) + ba_ref[...]
    m = jnp.max(lg, axis=1, keepdims=True)
    e = jnp.exp(lg - m)
    aw = e / jnp.dot(e, s_ref[...], preferred_element_type=jnp.float32)
    x = rpx * Ws + offx - 0.5
    y = rpy * Hs + offy - 0.5
    x0 = jnp.floor(x)
    fx = x - x0
    x1 = x0 + 1.0
    y0 = jnp.floor(y)
    fy = y - y0
    y1 = y0 + 1.0
    xs = jnp.concatenate([x0, x1, x0, x1], axis=1)
    ys = jnp.concatenate([y0, y0, y1, y1], axis=1)
    wxs = jnp.concatenate([1.0 - fx, fx, 1.0 - fx, fx], axis=1)
    wys = jnp.concatenate([1.0 - fy, 1.0 - fy, fy, fy], axis=1)
    aw4 = jnp.concatenate([aw, aw, aw, aw], axis=1)
    valid = (xs >= 0) & (xs <= Ws - 1) & (ys >= 0) & (ys <= Hs - 1)
    wgt_ref[0] = jnp.where(valid, aw4 * wxs * wys, 0.0)
    ixc = jnp.clip(xs, 0, Ws - 1).astype(jnp.int32)
    iyc = jnp.clip(ys, 0, Hs - 1).astype(jnp.int32)
    hlane = (lax.broadcasted_iota(jnp.int32, (TQ, NSAMP), 1) % (NH * NP)) // NP
    idx_ref[0] = b * (HW * NH) + (iyc * Ws + ixc) * NH + hlane


def _sampling_prep(query, rp, wox, woy, wa, bo2, ba2, smat, Hs, Ws, TQ):
    bs, nq, _ = query.shape
    HW = Hs * Ws
    body = functools.partial(_prep_body, Hs=Hs, Ws=Ws, HW=HW, TQ=TQ)
    hp = NH * NP
    return pl.pallas_call(
        body,
        grid=(bs, nq // TQ),
        in_specs=[
            pl.BlockSpec((1, TQ, EMBED), lambda b, t: (b, t, 0)),
            pl.BlockSpec((1, TQ, 2), lambda b, t: (b, t, 0)),
            pl.BlockSpec((EMBED, hp), lambda b, t: (0, 0)),
            pl.BlockSpec((EMBED, hp), lambda b, t: (0, 0)),
            pl.BlockSpec((EMBED, hp), lambda b, t: (0, 0)),
            pl.BlockSpec((2, hp), lambda b, t: (0, 0)),
            pl.BlockSpec((1, hp), lambda b, t: (0, 0)),
            pl.BlockSpec((hp, hp), lambda b, t: (0, 0)),
        ],
        out_specs=[
            pl.BlockSpec((1, TQ, NSAMP), lambda b, t: (b, t, 0)),
            pl.BlockSpec((1, TQ, NSAMP), lambda b, t: (b, t, 0)),
        ],
        out_shape=[
            jax.ShapeDtypeStruct((bs, nq, NSAMP), jnp.int32),
            jax.ShapeDtypeStruct((bs, nq, NSAMP), jnp.float32),
        ],
    )(query, rp, wox, woy, wa, bo2, ba2, smat)


def _bcast_lane(vec, s):
    return jnp.broadcast_to(lax.slice(vec, (s,), (s + 1,)), (LANES,))


def _sc_sample_combine(table, idx2, wgt2):
    BQ = idx2.shape[0]
    QW = BQ // NW
    C = 16
    NCH = QW // C
    GR = C * NSAMP // 128
    mesh = plsc.VectorSubcoreMesh(core_axis_name="c", subcore_axis_name="s")

    @functools.partial(
        pl.kernel,
        mesh=mesh,
        compiler_params=pltpu.CompilerParams(use_tc_tiling_on_sc=False,
                                             needs_layout_passes=False),
        out_type=jax.ShapeDtypeStruct((BQ, EMBED), jnp.float32),
        scratch_types=[
            pltpu.VMEM((C, NSAMP), jnp.int32),
            pltpu.VMEM((C, NSAMP), jnp.int32),
            pltpu.VMEM((C, NSAMP), jnp.float32),
            pltpu.VMEM((C, NSAMP), jnp.float32),
            pltpu.VMEM((C * NSAMP, HD // 2), jnp.int32),
            pltpu.VMEM((C * NSAMP, HD // 2), jnp.int32),
            pltpu.VMEM((C, EMBED), jnp.float32),
            pltpu.SemaphoreType.DMA,
            pltpu.SemaphoreType.DMA,
        ],
    )
    def k(table_r, idx_r, wgt_r, out_r, idx_v0, idx_v1, wgt_v0, wgt_v1,
          rows_v0, rows_v1, out_v, sem0, sem1):
        wid = lax.axis_index("s") * NC + lax.axis_index("c")
        base = wid * QW
        bufs = ((idx_v0, wgt_v0, rows_v0, sem0),
                (idx_v1, wgt_v1, rows_v1, sem1))

        def fire(jj, buf):
            idx_v, wgt_v, rows_v, sem = buf
            q0 = base + jj * C
            pltpu.sync_copy(idx_r.at[pl.ds(q0, C)], idx_v)
            pltpu.sync_copy(wgt_r.at[pl.ds(q0, C)], wgt_v)
            for g in range(GR):
                pltpu.async_copy(table_r.at[idx_v.at[g]],
                                 rows_v.at[pl.ds(g * 128, 128)], sem)

        def drain_compute_store(jj, buf):
            idx_v, wgt_v, rows_v, sem = buf
            for g in range(GR):
                pltpu.make_async_copy(table_r.at[idx_v.at[g]],
                                      rows_v.at[pl.ds(g * 128, 128)],
                                      sem).wait()

            def item_body(i, c2):
                isplat = jnp.broadcast_to(i, (LANES,)).astype(jnp.int32)
                for h in range(NH):
                    acc0 = jnp.zeros((LANES,), jnp.float32)
                    acc1 = jnp.zeros((LANES,), jnp.float32)
                    for c in range(NCORN):
                        for p in range(NP):
                            s = c * (NH * NP) + h * NP + p
                            ws = plsc.load_gather(
                                wgt_v,
                                [isplat,
                                 jnp.full((LANES,), s, jnp.int32)])
                            ri = rows_v[i * NSAMP + s]
                            lo = plsc.bitcast(
                                lax.shift_left(ri, 16), jnp.float32)
                            hi = plsc.bitcast(
                                ri & jnp.int32(-65536), jnp.float32)
                            acc0 = acc0 + ws * lo
                            acc1 = acc1 + ws * hi
                    out_v[i, pl.ds(h * HD, LANES)] = acc0
                    out_v[i, pl.ds(h * HD + LANES, LANES)] = acc1
                return c2

            lax.fori_loop(0, C, item_body, 0)
            pltpu.sync_copy(out_v, out_r.at[pl.ds(base + jj * C, C)])

        fire(0, bufs[0])

        def pair_body(t, carry):
            j0 = 2 * t
            fire(j0 + 1, bufs[1])
            drain_compute_store(j0, bufs[0])

            @pl.when(t < NCH // 2 - 1)
            def _():
                fire(j0 + 2, bufs[0])

            drain_compute_store(j0 + 1, bufs[1])
            return carry

        lax.fori_loop(0, NCH // 2, pair_body, 0)

    return k(table, idx2, wgt2)


def kernel(query, reference_points, value, Wv, bv, Wo, bo, Wa, ba, Wout,
           bout, spatial_shape):
    bs, nq, _ = query.shape
    HW = value.shape[1]
    Hs = int(math.isqrt(HW))
    Ws = HW // Hs

    wvt = Wv.T
    ar = jnp.arange(EMBED // 2, dtype=jnp.int32)
    lo_cols = (ar // LANES) * HD + ar % LANES
    hi_cols = lo_cols + LANES
    wlo, whi = wvt[:, lo_cols], wvt[:, hi_cols]
    blo = bv[lo_cols].reshape(1, -1)
    bhi = bv[hi_cols].reshape(1, -1)
    hp = NH * NP
    Wo4 = Wo.reshape(hp, 2, EMBED)
    wox = Wo4[:, 0, :].T
    woy = Wo4[:, 1, :].T
    bo2 = bo.reshape(hp, 2).T
    wa = Wa.T
    ba2 = ba.reshape(1, hp)
    gid = jnp.arange(hp, dtype=jnp.int32) // NP
    smat = (gid[:, None] == gid[None, :]).astype(jnp.float32)
    wout = Wout.T
    bout2 = bout.reshape(1, EMBED)

    ngroup = 2 if bs % 2 == 0 else 1
    gb = bs // ngroup
    outs = []
    for g in range(ngroup):
        sl = slice(g * gb, (g + 1) * gb)
        vp = _vproj_pack(value[sl].reshape(gb * HW, EMBED), wlo, whi,
                         blo, bhi, blk=1024)
        table = vp.reshape(gb * HW * NH, HD // 2)
        idx, wgt = _sampling_prep(query[sl], reference_points[sl], wox,
                                  woy, wa, bo2, ba2, smat, Hs, Ws, TQ=512)
        headout = _sc_sample_combine(table, idx.reshape(gb * nq, NSAMP),
                                     wgt.reshape(gb * nq, NSAMP))
        outs.append(_matmul_bias(headout, wout, bout2, blk=1024))
    out = outs[0] if ngroup == 1 else jnp.concatenate(outs, axis=0)
    return out.reshape(bs, nq, EMBED)
